# Initial kernel scaffold; baseline (speedup 1.0000x reference)
import jax
import jax.numpy as jnp
from jax.experimental import pallas as pl


def kernel(x, edge_index, edge_type, H, W, ln1_g, ln1_b, gate_W, gate_b, lin_W, lin_b, self_W, self_b, proj_W, proj_b, ln2_g, ln2_b, fc1_W, fc1_b, fc2_W, fc2_b):
    raise NotImplementedError("write your pallas kernel here")


# speedup vs baseline: 1.2448x; 1.2448x over previous
import functools

import jax
import jax.numpy as jnp
from jax import lax
from jax.experimental import pallas as pl
from jax.experimental.pallas import tpu as pltpu
from jax.experimental.pallas import tpu_sc as plsc

N = 10000
C = 128
R = 4
E = 320000
HID = 512
CA = 144
NT = 16
SB = 2048
NSB = 10
EPT = SB * NSB
E_PAD = EPT * NT
DUMP = 240
AGG_ROWS = N + DUMP
RPT = AGG_ROWS // NT


def _ln_aug_body(x_ref, g_ref, b_ref, o_ref):
    xv = x_ref[...]
    m = jnp.mean(xv, axis=-1, keepdims=True)
    var = jnp.mean((xv - m) ** 2, axis=-1, keepdims=True)
    h = (xv - m) * lax.rsqrt(var + 1e-5) * g_ref[...] + b_ref[...]
    o_ref[:, :C] = h
    col = lax.broadcasted_iota(jnp.int32, (x_ref.shape[0], CA - C), 1)
    o_ref[:, C:] = jnp.where(col == 0, 1.0, 0.0)


def _ln_aug(xf, g, b, bm=2000):
    grid = N // bm
    return pl.pallas_call(
        _ln_aug_body,
        grid=(grid,),
        in_specs=[
            pl.BlockSpec((bm, C), lambda i: (i, 0)),
            pl.BlockSpec((1, C), lambda i: (0, 0)),
            pl.BlockSpec((1, C), lambda i: (0, 0)),
        ],
        out_specs=pl.BlockSpec((bm, CA), lambda i: (i, 0)),
        out_shape=jax.ShapeDtypeStruct((N, CA), jnp.float32),
    )(xf, g.reshape(1, C), b.reshape(1, C))


def _sc_agg_kernel(h_hbm, src_hbm, idx_hbm, out_hbm,
                   src_v, idx_v, lidx_v, rows_v, agg_sh, sem):
    c = lax.axis_index("c")
    s = lax.axis_index("s")
    ebase = s * EPT
    lane = lax.iota(jnp.int32, 16)
    dump_v = N + jnp.remainder(s * 16 + lane, DUMP)
    zvec = jnp.zeros((16,), jnp.float32)

    for p in range(2):
        rbase = (c * 2 + p) * N

        def zero_body(i, carry):
            for j in range(CA // 16):
                rows_v[i, pl.ds(j * 16, 16)] = zvec
            return carry
        lax.fori_loop(0, 128, zero_body, 0)
        for j in range(RPT // 128):
            pltpu.sync_copy(rows_v, agg_sh.at[pl.ds(s * RPT + j * 128, 128)])
        plsc.subcore_barrier()

        def sb_body(sb, carry):
            pltpu.sync_copy(src_hbm.at[pl.ds(ebase + sb * SB, SB)], src_v)
            pltpu.sync_copy(idx_hbm.at[pl.ds(ebase + sb * SB, SB)], idx_v)

            def block_body(b, carry2):
                for v in range(8):
                    iv = idx_v[pl.ds(b * 128 + v * 16, 16)]
                    t = iv - rbase
                    m = (t >= 0) & (t < N)
                    lidx_v[0, pl.ds(v * 16, 16)] = jnp.where(m, t, dump_v)
                pltpu.async_copy(
                    h_hbm.at[src_v.at[pl.ds(b * 128, 128)]], rows_v, sem
                ).wait()
                pltpu.sync_copy(rows_v, agg_sh.at[lidx_v.at[0]], add=True)
                return carry2

            lax.fori_loop(0, SB // 128, block_body, 0)
            return carry

        lax.fori_loop(0, NSB, sb_body, 0)
        plsc.subcore_barrier()
        pltpu.sync_copy(
            agg_sh.at[pl.ds(s * RPT, RPT)],
            out_hbm.at[pl.ds((c * 2 + p) * AGG_ROWS + s * RPT, RPT)],
        )
        plsc.subcore_barrier()


def _sc_agg(h_aug, src_p, idx_p):
    mesh = plsc.VectorSubcoreMesh(core_axis_name="c", subcore_axis_name="s")
    kern = functools.partial(
        pl.kernel,
        mesh=mesh,
        compiler_params=pltpu.CompilerParams(use_tc_tiling_on_sc=False),
        out_type=jax.ShapeDtypeStruct((R * AGG_ROWS, CA), jnp.float32),
        scratch_types=[
            pltpu.VMEM((SB,), jnp.int32),
            pltpu.VMEM((SB,), jnp.int32),
            pltpu.VMEM((1, 128), jnp.int32),
            pltpu.VMEM((128, CA), jnp.float32),
            pltpu.VMEM_SHARED((AGG_ROWS, CA), jnp.float32),
            pltpu.SemaphoreType.DMA,
        ],
    )(_sc_agg_kernel)
    return kern(h_aug, src_p, idx_p)


def _main_body(x_ref, agg_ref, gw_ref, gb_ref, lw_ref, lsb_ref, sw_ref,
               pw_ref, pb_ref, g1_ref, b1_ref, g2_ref, b2_ref,
               f1w_ref, f1b_ref, f2w_ref, f2b_ref, o_ref):
    xv = x_ref[...]
    m = jnp.mean(xv, axis=-1, keepdims=True)
    var = jnp.mean((xv - m) ** 2, axis=-1, keepdims=True)
    h = (xv - m) * lax.rsqrt(var + 1e-5) * g1_ref[...] + b1_ref[...]
    gate = jax.nn.sigmoid(
        jnp.dot(h, gw_ref[...], preferred_element_type=jnp.float32)
        + gb_ref[...]
    )
    acc = jnp.dot(h, sw_ref[...], preferred_element_type=jnp.float32)
    for r in range(R):
        a = agg_ref[r]
        deg = a[:, C:C + 1]
        ar = a[:, :C] / jnp.maximum(deg, 1.0) * gate[:, r:r + 1]
        acc = acc + jnp.dot(ar, lw_ref[r], preferred_element_type=jnp.float32)
    out = jax.nn.gelu(acc + lsb_ref[...])
    out = jnp.dot(out, pw_ref[...], preferred_element_type=jnp.float32) + pb_ref[...]
    x2 = xv + out
    m2 = jnp.mean(x2, axis=-1, keepdims=True)
    var2 = jnp.mean((x2 - m2) ** 2, axis=-1, keepdims=True)
    h2 = (x2 - m2) * lax.rsqrt(var2 + 1e-5) * g2_ref[...] + b2_ref[...]
    h2 = jax.nn.gelu(
        jnp.dot(h2, f1w_ref[...], preferred_element_type=jnp.float32)
        + f1b_ref[...]
    )
    h2 = jnp.dot(h2, f2w_ref[...], preferred_element_type=jnp.float32) + f2b_ref[...]
    o_ref[...] = x2 + h2


def _main(xf, agg3, gate_W, gate_b, lin_W3, lin_self_b, self_W, proj_W,
          proj_b, ln1_g, ln1_b, ln2_g, ln2_b, fc1_W, fc1_b, fc2_W, fc2_b,
          bm=2000):
    grid = N // bm

    def full(shape):
        nd = len(shape)
        return pl.BlockSpec(shape, lambda i, _nd=nd: (0,) * _nd)

    return pl.pallas_call(
        _main_body,
        grid=(grid,),
        in_specs=[
            pl.BlockSpec((bm, C), lambda i: (i, 0)),
            pl.BlockSpec((R, bm, CA), lambda i: (0, i, 0)),
            full((C, R)),
            full((1, R)),
            full((R, C, C)),
            full((1, C)),
            full((C, C)),
            full((C, C)),
            full((1, C)),
            full((1, C)),
            full((1, C)),
            full((1, C)),
            full((1, C)),
            full((C, HID)),
            full((1, HID)),
            full((HID, C)),
            full((1, C)),
        ],
        out_specs=pl.BlockSpec((bm, C), lambda i: (i, 0)),
        out_shape=jax.ShapeDtypeStruct((N, C), jnp.float32),
    )(xf, agg3, gate_W, gate_b.reshape(1, R), lin_W3,
      lin_self_b.reshape(1, C), self_W, proj_W, proj_b.reshape(1, C), ln1_g.reshape(1, C),
      ln1_b.reshape(1, C), ln2_g.reshape(1, C), ln2_b.reshape(1, C),
      fc1_W, fc1_b.reshape(1, HID), fc2_W, fc2_b.reshape(1, C))


def kernel(x, edge_index, edge_type, H, W, ln1_g, ln1_b, gate_W, gate_b,
           lin_W, lin_b, self_W, self_b, proj_W, proj_b, ln2_g, ln2_b,
           fc1_W, fc1_b, fc2_W, fc2_b):
    xf = x.reshape(N, C)
    h_aug = _ln_aug(xf, ln1_g, ln1_b)

    src = edge_index[0]
    idx = edge_type.astype(jnp.int32) * N + edge_index[1]
    src_p = jnp.concatenate([src, jnp.zeros((E_PAD - E,), jnp.int32)])
    idx_p = jnp.concatenate(
        [idx, jnp.full((E_PAD - E,), jnp.int32(1 << 30))])

    agg_raw = _sc_agg(h_aug, src_p, idx_p)
    agg3 = agg_raw.reshape(R, AGG_ROWS, CA)

    lin_W3 = lin_W.reshape(R, C, C)
    out = _main(xf, agg3, gate_W, gate_b, lin_W3, lin_b + self_b, self_W,
                proj_W, proj_b, ln1_g, ln1_b, ln2_g, ln2_b, fc1_W, fc1_b,
                fc2_W, fc2_b)
    return out.reshape(1, N, C)


# speedup vs baseline: 6.1029x; 4.9026x over previous
import functools

import jax
import jax.numpy as jnp
from jax import lax
from jax.experimental import pallas as pl
from jax.experimental.pallas import tpu as pltpu
from jax.experimental.pallas import tpu_sc as plsc

N = 10000
C = 128
R = 4
E = 320000
HID = 512
CA = 144
NT = 16
SB = 2048
NSB = 10
EPT = SB * NSB
E_PAD = EPT * NT
DUMP = 240
AGG_ROWS = N + DUMP
RPT = AGG_ROWS // NT
CAP = SB + 256


def _ln_aug_body(x_ref, g_ref, b_ref, o_ref):
    xv = x_ref[...]
    m = jnp.mean(xv, axis=-1, keepdims=True)
    var = jnp.mean((xv - m) ** 2, axis=-1, keepdims=True)
    h = (xv - m) * lax.rsqrt(var + 1e-5) * g_ref[...] + b_ref[...]
    o_ref[:, :C] = h
    col = lax.broadcasted_iota(jnp.int32, (x_ref.shape[0], CA - C), 1)
    o_ref[:, C:] = jnp.where(col == 0, 1.0, 0.0)


def _ln_aug(xf, g, b, bm=2000):
    grid = N // bm
    return pl.pallas_call(
        _ln_aug_body,
        grid=(grid,),
        in_specs=[
            pl.BlockSpec((bm, C), lambda i: (i, 0)),
            pl.BlockSpec((1, C), lambda i: (0, 0)),
            pl.BlockSpec((1, C), lambda i: (0, 0)),
        ],
        out_specs=pl.BlockSpec((bm, CA), lambda i: (i, 0)),
        out_shape=jax.ShapeDtypeStruct((N, CA), jnp.float32),
    )(xf, g.reshape(1, C), b.reshape(1, C))


def _sc_agg_kernel(h_hbm, src_hbm, idx_hbm, out_hbm,
                   src_v, idx_v, csrc_v, clidx_v, lidx_v, rows_v, agg_sh, sem):
    c = lax.axis_index("c")
    s = lax.axis_index("s")
    ebase = s * EPT
    lane = lax.iota(jnp.int32, 16)
    dump_v = N + jnp.remainder(s * 16 + lane, DUMP)
    safe_v = s * 16 + lane
    zvec = jnp.zeros((16,), jnp.float32)

    def fire(nb_):
        def fire_body(b, carry2):
            for j in range(8):
                lidx_v[0, pl.ds(j * 16, 16)] = clidx_v[pl.ds(b * 128 + j * 16, 16)]
            pltpu.async_copy(
                h_hbm.at[csrc_v.at[pl.ds(b * 128, 128)]], rows_v, sem
            ).wait()
            pltpu.sync_copy(rows_v, agg_sh.at[lidx_v.at[0]], add=True)
            return carry2
        lax.fori_loop(0, nb_, fire_body, 0)

    for p in range(2):
        rbase = (c * 2 + p) * N

        def zero_body(i, carry):
            for j in range(CA // 16):
                rows_v[i, pl.ds(j * 16, 16)] = zvec
            return carry
        lax.fori_loop(0, 128, zero_body, 0)
        for j in range(RPT // 128):
            pltpu.sync_copy(rows_v, agg_sh.at[pl.ds(s * RPT + j * 128, 128)])
        plsc.subcore_barrier()

        def sb_body(sb, rem):
            pltpu.sync_copy(src_hbm.at[pl.ds(ebase + sb * SB, SB)], src_v)
            pltpu.sync_copy(idx_hbm.at[pl.ds(ebase + sb * SB, SB)], idx_v)

            def cvreg(v, cur):
                iv = idx_v[pl.ds(v * 16, 16)]
                sv = src_v[pl.ds(v * 16, 16)]
                t = iv - rbase
                m = (t >= 0) & (t < N)
                plsc.store_compressed(clidx_v.at[pl.ds(cur, 16)], t, mask=m)
                plsc.store_compressed(csrc_v.at[pl.ds(cur, 16)], sv, mask=m)
                return cur + jnp.sum(m.astype(jnp.int32))

            tot = lax.fori_loop(0, SB // 16, cvreg, rem)
            nb = tot // 128
            fire(nb)
            for j in range(8):
                t1 = clidx_v[pl.ds(nb * 128 + j * 16, 16)]
                t2 = csrc_v[pl.ds(nb * 128 + j * 16, 16)]
                clidx_v[pl.ds(j * 16, 16)] = t1
                csrc_v[pl.ds(j * 16, 16)] = t2
            return tot - nb * 128

        rem = lax.fori_loop(0, NSB, sb_body, jnp.int32(0))
        for j in range(8):
            mpad = (j * 16 + lane) >= rem
            t1 = clidx_v[pl.ds(j * 16, 16)]
            t2 = csrc_v[pl.ds(j * 16, 16)]
            clidx_v[pl.ds(j * 16, 16)] = jnp.where(mpad, dump_v, t1)
            csrc_v[pl.ds(j * 16, 16)] = jnp.where(mpad, safe_v, t2)
        fire(1)
        plsc.subcore_barrier()
        pltpu.sync_copy(
            agg_sh.at[pl.ds(s * RPT, RPT)],
            out_hbm.at[pl.ds((c * 2 + p) * AGG_ROWS + s * RPT, RPT)],
        )
        plsc.subcore_barrier()


def _sc_agg(h_aug, src_p, idx_p):
    mesh = plsc.VectorSubcoreMesh(core_axis_name="c", subcore_axis_name="s")
    kern = functools.partial(
        pl.kernel,
        mesh=mesh,
        compiler_params=pltpu.CompilerParams(
            use_tc_tiling_on_sc=False, needs_layout_passes=False),
        out_type=jax.ShapeDtypeStruct((R * AGG_ROWS, CA), jnp.float32),
        scratch_types=[
            pltpu.VMEM((SB,), jnp.int32),
            pltpu.VMEM((SB,), jnp.int32),
            pltpu.VMEM((CAP,), jnp.int32),
            pltpu.VMEM((CAP,), jnp.int32),
            pltpu.VMEM((1, 128), jnp.int32),
            pltpu.VMEM((128, CA), jnp.float32),
            pltpu.VMEM_SHARED((AGG_ROWS, CA), jnp.float32),
            pltpu.SemaphoreType.DMA,
        ],
    )(_sc_agg_kernel)
    return kern(h_aug, src_p, idx_p)


def _main_body(x_ref, agg_ref, gw_ref, gb_ref, lw_ref, lsb_ref, sw_ref,
               pw_ref, pb_ref, g1_ref, b1_ref, g2_ref, b2_ref,
               f1w_ref, f1b_ref, f2w_ref, f2b_ref, o_ref):
    xv = x_ref[...]
    m = jnp.mean(xv, axis=-1, keepdims=True)
    var = jnp.mean((xv - m) ** 2, axis=-1, keepdims=True)
    h = (xv - m) * lax.rsqrt(var + 1e-5) * g1_ref[...] + b1_ref[...]
    gate = jax.nn.sigmoid(
        jnp.dot(h, gw_ref[...], preferred_element_type=jnp.float32)
        + gb_ref[...]
    )
    acc = jnp.dot(h, sw_ref[...], preferred_element_type=jnp.float32)
    for r in range(R):
        a = agg_ref[r]
        deg = a[:, C:C + 1]
        ar = a[:, :C] / jnp.maximum(deg, 1.0) * gate[:, r:r + 1]
        acc = acc + jnp.dot(ar, lw_ref[r], preferred_element_type=jnp.float32)
    out = jax.nn.gelu(acc + lsb_ref[...])
    out = jnp.dot(out, pw_ref[...], preferred_element_type=jnp.float32) + pb_ref[...]
    x2 = xv + out
    m2 = jnp.mean(x2, axis=-1, keepdims=True)
    var2 = jnp.mean((x2 - m2) ** 2, axis=-1, keepdims=True)
    h2 = (x2 - m2) * lax.rsqrt(var2 + 1e-5) * g2_ref[...] + b2_ref[...]
    h2 = jax.nn.gelu(
        jnp.dot(h2, f1w_ref[...], preferred_element_type=jnp.float32)
        + f1b_ref[...]
    )
    h2 = jnp.dot(h2, f2w_ref[...], preferred_element_type=jnp.float32) + f2b_ref[...]
    o_ref[...] = x2 + h2


def _main(xf, agg3, gate_W, gate_b, lin_W3, lin_self_b, self_W, proj_W,
          proj_b, ln1_g, ln1_b, ln2_g, ln2_b, fc1_W, fc1_b, fc2_W, fc2_b,
          bm=2000):
    grid = N // bm

    def full(shape):
        nd = len(shape)
        return pl.BlockSpec(shape, lambda i, _nd=nd: (0,) * _nd)

    return pl.pallas_call(
        _main_body,
        grid=(grid,),
        in_specs=[
            pl.BlockSpec((bm, C), lambda i: (i, 0)),
            pl.BlockSpec((R, bm, CA), lambda i: (0, i, 0)),
            full((C, R)),
            full((1, R)),
            full((R, C, C)),
            full((1, C)),
            full((C, C)),
            full((C, C)),
            full((1, C)),
            full((1, C)),
            full((1, C)),
            full((1, C)),
            full((1, C)),
            full((C, HID)),
            full((1, HID)),
            full((HID, C)),
            full((1, C)),
        ],
        out_specs=pl.BlockSpec((bm, C), lambda i: (i, 0)),
        out_shape=jax.ShapeDtypeStruct((N, C), jnp.float32),
    )(xf, agg3, gate_W, gate_b.reshape(1, R), lin_W3,
      lin_self_b.reshape(1, C), self_W, proj_W, proj_b.reshape(1, C),
      ln1_g.reshape(1, C), ln1_b.reshape(1, C), ln2_g.reshape(1, C),
      ln2_b.reshape(1, C), fc1_W, fc1_b.reshape(1, HID), fc2_W,
      fc2_b.reshape(1, C))


def kernel(x, edge_index, edge_type, H, W, ln1_g, ln1_b, gate_W, gate_b,
           lin_W, lin_b, self_W, self_b, proj_W, proj_b, ln2_g, ln2_b,
           fc1_W, fc1_b, fc2_W, fc2_b):
    xf = x.reshape(N, C)
    h_aug = _ln_aug(xf, ln1_g, ln1_b)

    src = edge_index[0]
    idx = edge_type.astype(jnp.int32) * N + edge_index[1]
    src_p = jnp.concatenate([src, jnp.zeros((E_PAD - E,), jnp.int32)])
    idx_p = jnp.concatenate(
        [idx, jnp.full((E_PAD - E,), jnp.int32(1 << 30))])

    agg_raw = _sc_agg(h_aug, src_p, idx_p)
    agg3 = agg_raw.reshape(R, AGG_ROWS, CA)

    lin_W3 = lin_W.reshape(R, C, C)
    out = _main(xf, agg3, gate_W, gate_b, lin_W3, lin_b + self_b, self_W,
                proj_W, proj_b, ln1_g, ln1_b, ln2_g, ln2_b, fc1_W, fc1_b,
                fc2_W, fc2_b)
    return out.reshape(1, N, C)


# speedup vs baseline: 6.6576x; 1.0909x over previous
import functools

import jax
import jax.numpy as jnp
from jax import lax
from jax.experimental import pallas as pl
from jax.experimental.pallas import tpu as pltpu
from jax.experimental.pallas import tpu_sc as plsc

N = 10000
C = 128
R = 4
E = 320000
HID = 512
CA = 144
NT = 16
SB = 2048
NSB = 10
EPT = SB * NSB
E_PAD = EPT * NT
DUMP = 240
AGG_ROWS = N + DUMP
RPT = AGG_ROWS // NT
BK = 96
CAP = 2224


def _ln_aug_body(x_ref, g_ref, b_ref, o_ref):
    xv = x_ref[...]
    m = jnp.mean(xv, axis=-1, keepdims=True)
    var = jnp.mean((xv - m) ** 2, axis=-1, keepdims=True)
    h = (xv - m) * lax.rsqrt(var + 1e-5) * g_ref[...] + b_ref[...]
    o_ref[:, :C] = h
    col = lax.broadcasted_iota(jnp.int32, (x_ref.shape[0], CA - C), 1)
    o_ref[:, C:] = jnp.where(col == 0, 1.0, 0.0)


def _ln_aug(xf, g, b, bm=2000):
    grid = N // bm
    return pl.pallas_call(
        _ln_aug_body,
        grid=(grid,),
        in_specs=[
            pl.BlockSpec((bm, C), lambda i: (i, 0)),
            pl.BlockSpec((1, C), lambda i: (0, 0)),
            pl.BlockSpec((1, C), lambda i: (0, 0)),
        ],
        out_specs=pl.BlockSpec((bm, CA), lambda i: (i, 0)),
        out_shape=jax.ShapeDtypeStruct((N, CA), jnp.float32),
    )(xf, g.reshape(1, C), b.reshape(1, C))


def _sc_agg_kernel(h_hbm, src_hbm, idx_hbm, out_hbm,
                   src_v, idx_v, csrc_v, clidx_v, lidx_v, rows_v, agg_sh,
                   gsem0, gsem1):
    c = lax.axis_index("c")
    s = lax.axis_index("s")
    ebase = s * EPT
    lane = lax.iota(jnp.int32, 16)
    dump_v = N + jnp.remainder(s * 16 + lane, DUMP)
    safe_v = s * 16 + lane
    zvec = jnp.zeros((16,), jnp.float32)
    sems = (gsem0, gsem1)

    def issue(b, buf):
        for j in range(BK // 16):
            lidx_v[buf, pl.ds(j * 16, 16)] = clidx_v[pl.ds(b * BK + j * 16, 16)]
        pltpu.async_copy(
            h_hbm.at[csrc_v.at[pl.ds(b * BK, BK)]], rows_v.at[buf], sems[buf])

    def wait_gather(buf):
        pltpu.make_async_copy(
            h_hbm.at[csrc_v.at[pl.ds(0, BK)]], rows_v.at[buf], sems[buf]
        ).wait()

    def scatter(buf):
        pltpu.sync_copy(rows_v.at[buf], agg_sh.at[lidx_v.at[buf]], add=True)

    def fire(nb_):
        @pl.when(nb_ > 0)
        def _():
            issue(0, 0)

            def pair(k, carry2):
                b = 2 * k
                wait_gather(0)
                @pl.when(b + 1 < nb_)
                def _():
                    issue(b + 1, 1)
                scatter(0)
                @pl.when(b + 1 < nb_)
                def _():
                    wait_gather(1)
                    @pl.when(b + 2 < nb_)
                    def __():
                        issue(b + 2, 0)
                    scatter(1)
                return carry2

            lax.fori_loop(0, (nb_ + 1) // 2, pair, 0)

    for p in range(2):
        rbase = (c * 2 + p) * N

        def zero_body(i, carry):
            for j in range(CA // 16):
                rows_v[0, i, pl.ds(j * 16, 16)] = zvec
            return carry
        lax.fori_loop(0, BK, zero_body, 0)
        for j in range(RPT // BK):
            pltpu.sync_copy(rows_v.at[0],
                            agg_sh.at[pl.ds(s * RPT + j * BK, BK)])
        pltpu.sync_copy(rows_v.at[0, pl.ds(0, RPT % BK)],
                        agg_sh.at[pl.ds(s * RPT + (RPT // BK) * BK, RPT % BK)])
        plsc.subcore_barrier()

        def sb_body(sb, rem):
            pltpu.sync_copy(src_hbm.at[pl.ds(ebase + sb * SB, SB)], src_v)
            pltpu.sync_copy(idx_hbm.at[pl.ds(ebase + sb * SB, SB)], idx_v)

            def cvreg(v, cur):
                iv = idx_v[pl.ds(v * 16, 16)]
                sv = src_v[pl.ds(v * 16, 16)]
                t = iv - rbase
                m = (t >= 0) & (t < N)
                plsc.store_compressed(clidx_v.at[pl.ds(cur, 16)], t, mask=m)
                plsc.store_compressed(csrc_v.at[pl.ds(cur, 16)], sv, mask=m)
                return cur + jnp.sum(m.astype(jnp.int32))

            tot = lax.fori_loop(0, SB // 16, cvreg, rem)
            nb = tot // BK
            fire(nb)
            for j in range(BK // 16):
                t1 = clidx_v[pl.ds(nb * BK + j * 16, 16)]
                t2 = csrc_v[pl.ds(nb * BK + j * 16, 16)]
                clidx_v[pl.ds(j * 16, 16)] = t1
                csrc_v[pl.ds(j * 16, 16)] = t2
            return tot - nb * BK

        rem = lax.fori_loop(0, NSB, sb_body, jnp.int32(0))
        for j in range(BK // 16):
            mpad = (j * 16 + lane) >= rem
            t1 = clidx_v[pl.ds(j * 16, 16)]
            t2 = csrc_v[pl.ds(j * 16, 16)]
            clidx_v[pl.ds(j * 16, 16)] = jnp.where(mpad, dump_v, t1)
            csrc_v[pl.ds(j * 16, 16)] = jnp.where(mpad, safe_v, t2)
        fire(1)
        plsc.subcore_barrier()
        pltpu.sync_copy(
            agg_sh.at[pl.ds(s * RPT, RPT)],
            out_hbm.at[pl.ds((c * 2 + p) * AGG_ROWS + s * RPT, RPT)],
        )
        plsc.subcore_barrier()


def _sc_agg(h_aug, src_p, idx_p):
    mesh = plsc.VectorSubcoreMesh(core_axis_name="c", subcore_axis_name="s")
    kern = functools.partial(
        pl.kernel,
        mesh=mesh,
        compiler_params=pltpu.CompilerParams(
            use_tc_tiling_on_sc=False, needs_layout_passes=False),
        out_type=jax.ShapeDtypeStruct((R * AGG_ROWS, CA), jnp.float32),
        scratch_types=[
            pltpu.VMEM((SB,), jnp.int32),
            pltpu.VMEM((SB,), jnp.int32),
            pltpu.VMEM((CAP,), jnp.int32),
            pltpu.VMEM((CAP,), jnp.int32),
            pltpu.VMEM((2, BK), jnp.int32),
            pltpu.VMEM((2, BK, CA), jnp.float32),
            pltpu.VMEM_SHARED((AGG_ROWS, CA), jnp.float32),
            pltpu.SemaphoreType.DMA,
            pltpu.SemaphoreType.DMA,
        ],
    )(_sc_agg_kernel)
    return kern(h_aug, src_p, idx_p)


def _main_body(x_ref, agg_ref, gw_ref, gb_ref, lw_ref, lsb_ref, sw_ref,
               pw_ref, pb_ref, g1_ref, b1_ref, g2_ref, b2_ref,
               f1w_ref, f1b_ref, f2w_ref, f2b_ref, o_ref):
    xv = x_ref[...]
    m = jnp.mean(xv, axis=-1, keepdims=True)
    var = jnp.mean((xv - m) ** 2, axis=-1, keepdims=True)
    h = (xv - m) * lax.rsqrt(var + 1e-5) * g1_ref[...] + b1_ref[...]
    gate = jax.nn.sigmoid(
        jnp.dot(h, gw_ref[...], preferred_element_type=jnp.float32)
        + gb_ref[...]
    )
    acc = jnp.dot(h, sw_ref[...], preferred_element_type=jnp.float32)
    for r in range(R):
        a = agg_ref[r]
        deg = a[:, C:C + 1]
        ar = a[:, :C] / jnp.maximum(deg, 1.0) * gate[:, r:r + 1]
        acc = acc + jnp.dot(ar, lw_ref[r], preferred_element_type=jnp.float32)
    out = jax.nn.gelu(acc + lsb_ref[...])
    out = jnp.dot(out, pw_ref[...], preferred_element_type=jnp.float32) + pb_ref[...]
    x2 = xv + out
    m2 = jnp.mean(x2, axis=-1, keepdims=True)
    var2 = jnp.mean((x2 - m2) ** 2, axis=-1, keepdims=True)
    h2 = (x2 - m2) * lax.rsqrt(var2 + 1e-5) * g2_ref[...] + b2_ref[...]
    h2 = jax.nn.gelu(
        jnp.dot(h2, f1w_ref[...], preferred_element_type=jnp.float32)
        + f1b_ref[...]
    )
    h2 = jnp.dot(h2, f2w_ref[...], preferred_element_type=jnp.float32) + f2b_ref[...]
    o_ref[...] = x2 + h2


def _main(xf, agg3, gate_W, gate_b, lin_W3, lin_self_b, self_W, proj_W,
          proj_b, ln1_g, ln1_b, ln2_g, ln2_b, fc1_W, fc1_b, fc2_W, fc2_b,
          bm=2000):
    grid = N // bm

    def full(shape):
        nd = len(shape)
        return pl.BlockSpec(shape, lambda i, _nd=nd: (0,) * _nd)

    return pl.pallas_call(
        _main_body,
        grid=(grid,),
        in_specs=[
            pl.BlockSpec((bm, C), lambda i: (i, 0)),
            pl.BlockSpec((R, bm, CA), lambda i: (0, i, 0)),
            full((C, R)),
            full((1, R)),
            full((R, C, C)),
            full((1, C)),
            full((C, C)),
            full((C, C)),
            full((1, C)),
            full((1, C)),
            full((1, C)),
            full((1, C)),
            full((1, C)),
            full((C, HID)),
            full((1, HID)),
            full((HID, C)),
            full((1, C)),
        ],
        out_specs=pl.BlockSpec((bm, C), lambda i: (i, 0)),
        out_shape=jax.ShapeDtypeStruct((N, C), jnp.float32),
    )(xf, agg3, gate_W, gate_b.reshape(1, R), lin_W3,
      lin_self_b.reshape(1, C), self_W, proj_W, proj_b.reshape(1, C),
      ln1_g.reshape(1, C), ln1_b.reshape(1, C), ln2_g.reshape(1, C),
      ln2_b.reshape(1, C), fc1_W, fc1_b.reshape(1, HID), fc2_W,
      fc2_b.reshape(1, C))


def kernel(x, edge_index, edge_type, H, W, ln1_g, ln1_b, gate_W, gate_b,
           lin_W, lin_b, self_W, self_b, proj_W, proj_b, ln2_g, ln2_b,
           fc1_W, fc1_b, fc2_W, fc2_b):
    xf = x.reshape(N, C)
    h_aug = _ln_aug(xf, ln1_g, ln1_b)

    src = edge_index[0]
    idx = edge_type.astype(jnp.int32) * N + edge_index[1]
    src_p = jnp.concatenate([src, jnp.zeros((E_PAD - E,), jnp.int32)])
    idx_p = jnp.concatenate(
        [idx, jnp.full((E_PAD - E,), jnp.int32(1 << 30))])

    agg_raw = _sc_agg(h_aug, src_p, idx_p)
    agg3 = agg_raw.reshape(R, AGG_ROWS, CA)

    lin_W3 = lin_W.reshape(R, C, C)
    out = _main(xf, agg3, gate_W, gate_b, lin_W3, lin_b + self_b, self_W,
                proj_W, proj_b, ln1_g, ln1_b, ln2_g, ln2_b, fc1_W, fc1_b,
                fc2_W, fc2_b)
    return out.reshape(1, N, C)


# speedup vs baseline: 7.1627x; 1.0759x over previous
import functools

import jax
import jax.numpy as jnp
from jax import lax
from jax.experimental import pallas as pl
from jax.experimental.pallas import tpu as pltpu
from jax.experimental.pallas import tpu_sc as plsc

N = 10000
C = 128
R = 4
E = 320000
HID = 512
NT = 16
SB = 2048
NSB = 10
EPT = SB * NSB
E_PAD = EPT * NT
DUMP = 240
AGG_ROWS = N + DUMP
RPT = AGG_ROWS // NT
BK = 64
CAP = 2176
PADV = (3 << 28) | (16383 << 14)


def _ln_body(x_ref, g_ref, b_ref, o_ref):
    xv = x_ref[...]
    m = jnp.mean(xv, axis=-1, keepdims=True)
    var = jnp.mean((xv - m) ** 2, axis=-1, keepdims=True)
    o_ref[...] = (xv - m) * lax.rsqrt(var + 1e-5) * g_ref[...] + b_ref[...]


def _ln(xf, g, b, bm=2000):
    return pl.pallas_call(
        _ln_body,
        grid=(N // bm,),
        in_specs=[
            pl.BlockSpec((bm, C), lambda i: (i, 0)),
            pl.BlockSpec((1, C), lambda i: (0, 0)),
            pl.BlockSpec((1, C), lambda i: (0, 0)),
        ],
        out_specs=pl.BlockSpec((bm, C), lambda i: (i, 0)),
        out_shape=jax.ShapeDtypeStruct((N, C), jnp.float32),
    )(xf, g.reshape(1, C), b.reshape(1, C))


def _sc_agg_kernel(h_hbm, pk_hbm, out_hbm,
                   pk_v, csrc_v, clidx_v, lidx_v, gsrc_v, rows_v, ldeg_v,
                   dacc_v, dtmp_v, agg_sh, degp_sh,
                   gsem0, gsem1, psem):
    c = lax.axis_index("c")
    s = lax.axis_index("s")
    ebase = s * EPT
    lane = lax.iota(jnp.int32, 16)
    dump_v = N + jnp.remainder(s * 16 + lane, DUMP)
    safe_v = s * 16 + lane
    zvec = jnp.zeros((16,), jnp.float32)
    sems = (gsem0, gsem1)

    def issue(b, buf):
        for j in range(BK // 16):
            tv = clidx_v[pl.ds(b * BK + j * 16, 16)]
            sv = csrc_v[pl.ds(b * BK + j * 16, 16)]
            lidx_v[buf, pl.ds(j * 16, 16)] = tv
            gsrc_v[buf, pl.ds(j * 16, 16)] = sv
            cnts, lastm = plsc.scan_count(tv)
            plsc.addupdate_scatter(
                ldeg_v, [tv], cnts.astype(jnp.float32), mask=lastm)
        pltpu.async_copy(
            h_hbm.at[gsrc_v.at[buf]], rows_v.at[buf], sems[buf])

    def wait_gather(buf):
        pltpu.make_async_copy(
            h_hbm.at[gsrc_v.at[buf]], rows_v.at[buf], sems[buf]).wait()

    def scatter(buf):
        pltpu.sync_copy(rows_v.at[buf], agg_sh.at[lidx_v.at[buf]], add=True)

    def fire(nb_):
        @pl.when(nb_ > 0)
        def _():
            issue(0, 0)

            def pair(k, carry2):
                b = 2 * k
                wait_gather(0)
                @pl.when(b + 1 < nb_)
                def _():
                    issue(b + 1, 1)
                scatter(0)
                @pl.when(b + 1 < nb_)
                def _():
                    wait_gather(1)
                    @pl.when(b + 2 < nb_)
                    def __():
                        issue(b + 2, 0)
                    scatter(1)
                return carry2

            lax.fori_loop(0, (nb_ + 1) // 2, pair, 0)

    for p in range(2):
        rbase = (c * 2 + p) * N

        def zero_body(i, carry):
            for j in range(C // 16):
                rows_v[0, i, pl.ds(j * 16, 16)] = zvec
            return carry
        lax.fori_loop(0, BK, zero_body, 0)

        def ldeg_zero(i, carry):
            ldeg_v[pl.ds(i * 16, 16)] = zvec
            return carry
        lax.fori_loop(0, AGG_ROWS // 16, ldeg_zero, 0)

        for j in range(RPT // BK):
            pltpu.sync_copy(rows_v.at[0],
                            agg_sh.at[pl.ds(s * RPT + j * BK, BK)])
        plsc.subcore_barrier()

        pltpu.async_copy(pk_hbm.at[pl.ds(ebase, SB)], pk_v.at[0], psem)

        def sb_body(sb, rem):
            buf = sb % 2
            pltpu.make_async_copy(
                pk_hbm.at[pl.ds(ebase, SB)], pk_v.at[buf], psem).wait()

            @pl.when(sb + 1 < NSB)
            def _():
                pltpu.async_copy(
                    pk_hbm.at[pl.ds(ebase + (sb + 1) * SB, SB)],
                    pk_v.at[1 - buf], psem)

            def cvreg(v, cur):
                pk = pk_v[buf, pl.ds(v * 16, 16)]
                sv = pk & 0x3FFF
                dv = (pk >> 14) & 0x3FFF
                ev = pk >> 28
                t = ev * N + dv - rbase
                m = (t >= 0) & (t < N)
                plsc.store_compressed(clidx_v.at[pl.ds(cur, 16)], t, mask=m)
                plsc.store_compressed(csrc_v.at[pl.ds(cur, 16)], sv, mask=m)
                return cur + jnp.sum(m.astype(jnp.int32))

            tot = lax.fori_loop(0, SB // 16, cvreg, rem)
            nb = tot // BK
            fire(nb)
            for j in range(BK // 16):
                t1 = clidx_v[pl.ds(nb * BK + j * 16, 16)]
                t2 = csrc_v[pl.ds(nb * BK + j * 16, 16)]
                clidx_v[pl.ds(j * 16, 16)] = t1
                csrc_v[pl.ds(j * 16, 16)] = t2
            return tot - nb * BK

        rem = lax.fori_loop(0, NSB, sb_body, jnp.int32(0))
        for j in range(BK // 16):
            mpad = (j * 16 + lane) >= rem
            t1 = clidx_v[pl.ds(j * 16, 16)]
            t2 = csrc_v[pl.ds(j * 16, 16)]
            clidx_v[pl.ds(j * 16, 16)] = jnp.where(mpad, dump_v, t1)
            csrc_v[pl.ds(j * 16, 16)] = jnp.where(mpad, safe_v, t2)
        fire(1)
        pltpu.sync_copy(ldeg_v, degp_sh.at[s])
        plsc.subcore_barrier()

        pltpu.sync_copy(degp_sh.at[0, pl.ds(s * RPT, RPT)], dacc_v)
        for q in range(1, NT):
            pltpu.sync_copy(degp_sh.at[q, pl.ds(s * RPT, RPT)], dtmp_v)
            def dadd(i, carry):
                dacc_v[pl.ds(i * 16, 16)] = (
                    dacc_v[pl.ds(i * 16, 16)] + dtmp_v[pl.ds(i * 16, 16)])
                return carry
            lax.fori_loop(0, RPT // 16, dadd, 0)

        def nbody(jj, carry):
            j0 = jj * BK
            pltpu.sync_copy(agg_sh.at[pl.ds(s * RPT + j0, BK)],
                            rows_v.at[0])
            for g in range(BK // 16):
                dvec = dacc_v[pl.ds(j0 + g * 16, 16)]
                inv = 1.0 / jnp.maximum(dvec, 1.0)
                for r16 in range(16):
                    sc = jnp.sum(jnp.where(lane == r16, inv, 0.0))
                    row = g * 16 + r16
                    for j in range(C // 16):
                        rows_v[0, row, pl.ds(j * 16, 16)] = (
                            rows_v[0, row, pl.ds(j * 16, 16)] * sc)
            pltpu.sync_copy(
                rows_v.at[0],
                out_hbm.at[c * 2 + p, pl.ds(s * RPT + j0, BK)])
            return carry
        lax.fori_loop(0, RPT // BK, nbody, 0)
        plsc.subcore_barrier()


def _sc_agg(h, pk_p):
    mesh = plsc.VectorSubcoreMesh(core_axis_name="c", subcore_axis_name="s")
    kern = functools.partial(
        pl.kernel,
        mesh=mesh,
        compiler_params=pltpu.CompilerParams(needs_layout_passes=False),
        out_type=jax.ShapeDtypeStruct((R, AGG_ROWS, C), jnp.float32),
        scratch_types=[
            pltpu.VMEM((2, SB), jnp.int32),
            pltpu.VMEM((CAP,), jnp.int32),
            pltpu.VMEM((CAP,), jnp.int32),
            pltpu.VMEM((2, BK), jnp.int32),
            pltpu.VMEM((2, BK), jnp.int32),
            pltpu.VMEM((2, BK, C), jnp.float32),
            pltpu.VMEM((AGG_ROWS,), jnp.float32),
            pltpu.VMEM((RPT,), jnp.float32),
            pltpu.VMEM((RPT,), jnp.float32),
            pltpu.VMEM_SHARED((AGG_ROWS, C), jnp.float32),
            pltpu.VMEM_SHARED((NT, AGG_ROWS), jnp.float32),
            pltpu.SemaphoreType.DMA,
            pltpu.SemaphoreType.DMA,
            pltpu.SemaphoreType.DMA,
        ],
    )(_sc_agg_kernel)
    return kern(h, pk_p)


def _main_body(x_ref, agg_ref, gw_ref, gb_ref, lw_ref, lsb_ref, sw_ref,
               pw_ref, pb_ref, g1_ref, b1_ref, g2_ref, b2_ref,
               f1w_ref, f1b_ref, f2w_ref, f2b_ref, o_ref):
    xv = x_ref[...]
    m = jnp.mean(xv, axis=-1, keepdims=True)
    var = jnp.mean((xv - m) ** 2, axis=-1, keepdims=True)
    h = (xv - m) * lax.rsqrt(var + 1e-5) * g1_ref[...] + b1_ref[...]
    gate = jax.nn.sigmoid(
        jnp.dot(h, gw_ref[...], preferred_element_type=jnp.float32)
        + gb_ref[...]
    )
    acc = jnp.dot(h, sw_ref[...], preferred_element_type=jnp.float32)
    for r in range(R):
        ar = agg_ref[r] * gate[:, r:r + 1]
        acc = acc + jnp.dot(ar, lw_ref[r], preferred_element_type=jnp.float32)
    out = jax.nn.gelu(acc + lsb_ref[...])
    out = jnp.dot(out, pw_ref[...], preferred_element_type=jnp.float32) + pb_ref[...]
    x2 = xv + out
    m2 = jnp.mean(x2, axis=-1, keepdims=True)
    var2 = jnp.mean((x2 - m2) ** 2, axis=-1, keepdims=True)
    h2 = (x2 - m2) * lax.rsqrt(var2 + 1e-5) * g2_ref[...] + b2_ref[...]
    h2 = jax.nn.gelu(
        jnp.dot(h2, f1w_ref[...], preferred_element_type=jnp.float32)
        + f1b_ref[...]
    )
    h2 = jnp.dot(h2, f2w_ref[...], preferred_element_type=jnp.float32) + f2b_ref[...]
    o_ref[...] = x2 + h2


def _main(xf, agg3, gate_W, gate_b, lin_W3, lin_self_b, self_W, proj_W,
          proj_b, ln1_g, ln1_b, ln2_g, ln2_b, fc1_W, fc1_b, fc2_W, fc2_b,
          bm=2000):
    grid = N // bm

    def full(shape):
        nd = len(shape)
        return pl.BlockSpec(shape, lambda i, _nd=nd: (0,) * _nd)

    return pl.pallas_call(
        _main_body,
        grid=(grid,),
        in_specs=[
            pl.BlockSpec((bm, C), lambda i: (i, 0)),
            pl.BlockSpec((R, bm, C), lambda i: (0, i, 0)),
            full((C, R)),
            full((1, R)),
            full((R, C, C)),
            full((1, C)),
            full((C, C)),
            full((C, C)),
            full((1, C)),
            full((1, C)),
            full((1, C)),
            full((1, C)),
            full((1, C)),
            full((C, HID)),
            full((1, HID)),
            full((HID, C)),
            full((1, C)),
        ],
        out_specs=pl.BlockSpec((bm, C), lambda i: (i, 0)),
        out_shape=jax.ShapeDtypeStruct((N, C), jnp.float32),
    )(xf, agg3, gate_W, gate_b.reshape(1, R), lin_W3,
      lin_self_b.reshape(1, C), self_W, proj_W, proj_b.reshape(1, C),
      ln1_g.reshape(1, C), ln1_b.reshape(1, C), ln2_g.reshape(1, C),
      ln2_b.reshape(1, C), fc1_W, fc1_b.reshape(1, HID), fc2_W,
      fc2_b.reshape(1, C))


def kernel(x, edge_index, edge_type, H, W, ln1_g, ln1_b, gate_W, gate_b,
           lin_W, lin_b, self_W, self_b, proj_W, proj_b, ln2_g, ln2_b,
           fc1_W, fc1_b, fc2_W, fc2_b):
    xf = x.reshape(N, C)
    h = _ln(xf, ln1_g, ln1_b)

    pk = (edge_index[0]
          | (edge_index[1] << 14)
          | (edge_type.astype(jnp.int32) << 28))
    pk_p = jnp.concatenate(
        [pk, jnp.full((E_PAD - E,), jnp.int32(PADV))])

    agg3 = _sc_agg(h, pk_p)

    lin_W3 = lin_W.reshape(R, C, C)
    out = _main(xf, agg3, gate_W, gate_b, lin_W3, lin_b + self_b, self_W,
                proj_W, proj_b, ln1_g, ln1_b, ln2_g, ln2_b, fc1_W, fc1_b,
                fc2_W, fc2_b)
    return out.reshape(1, N, C)


# speedup vs baseline: 8.2506x; 1.1519x over previous
import functools

import jax
import jax.numpy as jnp
from jax import lax
from jax.experimental import pallas as pl
from jax.experimental.pallas import tpu as pltpu
from jax.experimental.pallas import tpu_sc as plsc

N = 10000
C = 128
R = 4
E = 320000
HID = 512
NT = 16
SB = 2048
NSB = 10
EPT = SB * NSB
E_PAD = EPT * NT
DUMP = 240
AGG_ROWS = N + DUMP
RPT = AGG_ROWS // NT
BK = 96
CAP = 2208
DGR = AGG_ROWS // 128
PADV = (3 << 28) | (16383 << 14)


def _ln_body(x_ref, g_ref, b_ref, o_ref):
    xv = x_ref[...]
    m = jnp.mean(xv, axis=-1, keepdims=True)
    var = jnp.mean((xv - m) ** 2, axis=-1, keepdims=True)
    o_ref[...] = (xv - m) * lax.rsqrt(var + 1e-5) * g_ref[...] + b_ref[...]


def _ln(xf, g, b, bm=2000):
    return pl.pallas_call(
        _ln_body,
        grid=(N // bm,),
        in_specs=[
            pl.BlockSpec((bm, C), lambda i: (i, 0)),
            pl.BlockSpec((1, C), lambda i: (0, 0)),
            pl.BlockSpec((1, C), lambda i: (0, 0)),
        ],
        out_specs=pl.BlockSpec((bm, C), lambda i: (i, 0)),
        out_shape=jax.ShapeDtypeStruct((N, C), jnp.float32),
    )(xf, g.reshape(1, C), b.reshape(1, C))


def _sc_agg_kernel(h_hbm, pk_hbm, out_hbm,
                   pk_v, csrc_v, clidx_v, lidx_v, gsrc_v, rows_v, ldeg_v,
                   diota_v, gidx_v, dacc_v, agg_sh, deg_sh,
                   gsem0, gsem1, psem):
    c = lax.axis_index("c")
    s = lax.axis_index("s")
    ebase = s * EPT
    lane = lax.iota(jnp.int32, 16)
    dump_v = N + jnp.remainder(s * 16 + lane, DUMP)
    safe_v = s * 16 + lane
    zvec = jnp.zeros((16,), jnp.float32)
    sems = (gsem0, gsem1)

    for v in range(DGR // 16):
        diota_v[0, pl.ds(v * 16, 16)] = v * 16 + lane
    gidx_v[0, pl.ds(0, 16)] = s * (DGR // NT) + jnp.minimum(lane, DGR // NT - 1)

    def issue(b, buf):
        for j in range(BK // 16):
            tv = clidx_v[pl.ds(b * BK + j * 16, 16)]
            sv = csrc_v[pl.ds(b * BK + j * 16, 16)]
            lidx_v[buf, pl.ds(j * 16, 16)] = tv
            gsrc_v[buf, pl.ds(j * 16, 16)] = sv
            cnts, lastm = plsc.scan_count(tv)
            plsc.addupdate_scatter(
                ldeg_v, [tv >> 7, tv & 127], cnts.astype(jnp.float32),
                mask=lastm)
        pltpu.async_copy(
            h_hbm.at[gsrc_v.at[buf]], rows_v.at[buf], sems[buf])

    def wait_gather(buf):
        pltpu.make_async_copy(
            h_hbm.at[gsrc_v.at[buf]], rows_v.at[buf], sems[buf]).wait()

    def scatter(buf):
        pltpu.sync_copy(rows_v.at[buf], agg_sh.at[lidx_v.at[buf]], add=True)

    def fire(nb_):
        @pl.when(nb_ > 0)
        def _():
            issue(0, 0)

            def pair(k, carry2):
                b = 2 * k
                wait_gather(0)
                @pl.when(b + 1 < nb_)
                def _():
                    issue(b + 1, 1)
                scatter(0)
                @pl.when(b + 1 < nb_)
                def _():
                    wait_gather(1)
                    @pl.when(b + 2 < nb_)
                    def __():
                        issue(b + 2, 0)
                    scatter(1)
                return carry2

            lax.fori_loop(0, (nb_ + 1) // 2, pair, 0)

    for p in range(2):
        rbase = (c * 2 + p) * N

        def zero_body(i, carry):
            for j in range(C // 16):
                rows_v[0, i, pl.ds(j * 16, 16)] = zvec
            return carry
        lax.fori_loop(0, BK, zero_body, 0)

        def ldeg_zero(i, carry):
            for j in range(C // 16):
                ldeg_v[i, pl.ds(j * 16, 16)] = zvec
            return carry
        lax.fori_loop(0, DGR, ldeg_zero, 0)

        for j in range(RPT // BK):
            pltpu.sync_copy(rows_v.at[0],
                            agg_sh.at[pl.ds(s * RPT + j * BK, BK)])
        pltpu.sync_copy(
            rows_v.at[0, pl.ds(0, RPT % BK)],
            agg_sh.at[pl.ds(s * RPT + (RPT // BK) * BK, RPT % BK)])
        @pl.when(s < DGR // 8)
        def _():
            pltpu.sync_copy(rows_v.at[0, pl.ds(0, 8)],
                            deg_sh.at[pl.ds(s * 8, 8)])
        plsc.subcore_barrier()

        pltpu.async_copy(pk_hbm.at[pl.ds(ebase, SB)], pk_v.at[0], psem)

        def sb_body(sb, rem):
            buf = sb % 2
            pltpu.make_async_copy(
                pk_hbm.at[pl.ds(ebase, SB)], pk_v.at[buf], psem).wait()

            @pl.when(sb + 1 < NSB)
            def _():
                pltpu.async_copy(
                    pk_hbm.at[pl.ds(ebase + (sb + 1) * SB, SB)],
                    pk_v.at[1 - buf], psem)

            def cvreg(v, cur):
                pk = pk_v[buf, pl.ds(v * 16, 16)]
                sv = pk & 0x3FFF
                dv = (pk >> 14) & 0x3FFF
                ev = pk >> 28
                t = ev * N + dv - rbase
                m = (t >= 0) & (t < N)
                plsc.store_compressed(clidx_v.at[pl.ds(cur, 16)], t, mask=m)
                plsc.store_compressed(csrc_v.at[pl.ds(cur, 16)], sv, mask=m)
                return cur + jnp.sum(m.astype(jnp.int32))

            tot = lax.fori_loop(0, SB // 16, cvreg, rem)
            nb = tot // BK
            fire(nb)
            for j in range(BK // 16):
                t1 = clidx_v[pl.ds(nb * BK + j * 16, 16)]
                t2 = csrc_v[pl.ds(nb * BK + j * 16, 16)]
                clidx_v[pl.ds(j * 16, 16)] = t1
                csrc_v[pl.ds(j * 16, 16)] = t2
            return tot - nb * BK

        rem = lax.fori_loop(0, NSB, sb_body, jnp.int32(0))
        for j in range(BK // 16):
            mpad = (j * 16 + lane) >= rem
            t1 = clidx_v[pl.ds(j * 16, 16)]
            t2 = csrc_v[pl.ds(j * 16, 16)]
            clidx_v[pl.ds(j * 16, 16)] = jnp.where(mpad, dump_v, t1)
            csrc_v[pl.ds(j * 16, 16)] = jnp.where(mpad, safe_v, t2)
        fire(1)
        pltpu.sync_copy(ldeg_v, deg_sh.at[diota_v.at[0]], add=True)
        plsc.subcore_barrier()

        pltpu.async_copy(deg_sh.at[gidx_v.at[0]], dacc_v, psem).wait()

        def norm_chunk(j0, nrows):
            pltpu.sync_copy(agg_sh.at[pl.ds(s * RPT + j0, nrows)],
                            rows_v.at[0, pl.ds(0, nrows)])
            for g in range(nrows // 16):
                w = j0 + g * 16
                dvec = dacc_v[w // 128, pl.ds(w % 128, 16)]
                inv = 1.0 / jnp.maximum(dvec, 1.0)
                for r16 in range(16):
                    sc = jnp.sum(jnp.where(lane == r16, inv, 0.0))
                    row = g * 16 + r16
                    for j in range(C // 16):
                        rows_v[0, row, pl.ds(j * 16, 16)] = (
                            rows_v[0, row, pl.ds(j * 16, 16)] * sc)
            pltpu.sync_copy(
                rows_v.at[0, pl.ds(0, nrows)],
                out_hbm.at[c * 2 + p, pl.ds(s * RPT + j0, nrows)])

        def nbody(jj, carry):
            norm_chunk(jj * BK, BK)
            return carry
        lax.fori_loop(0, RPT // BK, nbody, 0)
        norm_chunk((RPT // BK) * BK, RPT % BK)
        plsc.subcore_barrier()


def _sc_agg(h, pk_p):
    mesh = plsc.VectorSubcoreMesh(core_axis_name="c", subcore_axis_name="s")
    kern = functools.partial(
        pl.kernel,
        mesh=mesh,
        compiler_params=pltpu.CompilerParams(needs_layout_passes=False),
        out_type=jax.ShapeDtypeStruct((R, AGG_ROWS, C), jnp.float32),
        scratch_types=[
            pltpu.VMEM((2, SB), jnp.int32),
            pltpu.VMEM((CAP,), jnp.int32),
            pltpu.VMEM((CAP,), jnp.int32),
            pltpu.VMEM((2, BK), jnp.int32),
            pltpu.VMEM((2, BK), jnp.int32),
            pltpu.VMEM((2, BK, C), jnp.float32),
            pltpu.VMEM((DGR, C), jnp.float32),
            pltpu.VMEM((1, DGR), jnp.int32),
            pltpu.VMEM((1, 16), jnp.int32),
            pltpu.VMEM((16, C), jnp.float32),
            pltpu.VMEM_SHARED((AGG_ROWS, C), jnp.float32),
            pltpu.VMEM_SHARED((DGR, C), jnp.float32),
            pltpu.SemaphoreType.DMA,
            pltpu.SemaphoreType.DMA,
            pltpu.SemaphoreType.DMA,
        ],
    )(_sc_agg_kernel)
    return kern(h, pk_p)


def _main_body(x_ref, agg_ref, gw_ref, gb_ref, lw_ref, lsb_ref, sw_ref,
               pw_ref, pb_ref, g1_ref, b1_ref, g2_ref, b2_ref,
               f1w_ref, f1b_ref, f2w_ref, f2b_ref, o_ref):
    xv = x_ref[...]
    m = jnp.mean(xv, axis=-1, keepdims=True)
    var = jnp.mean((xv - m) ** 2, axis=-1, keepdims=True)
    h = (xv - m) * lax.rsqrt(var + 1e-5) * g1_ref[...] + b1_ref[...]
    gate = jax.nn.sigmoid(
        jnp.dot(h, gw_ref[...], preferred_element_type=jnp.float32)
        + gb_ref[...]
    )
    acc = jnp.dot(h, sw_ref[...], preferred_element_type=jnp.float32)
    for r in range(R):
        ar = agg_ref[r] * gate[:, r:r + 1]
        acc = acc + jnp.dot(ar, lw_ref[r], preferred_element_type=jnp.float32)
    out = jax.nn.gelu(acc + lsb_ref[...])
    out = jnp.dot(out, pw_ref[...], preferred_element_type=jnp.float32) + pb_ref[...]
    x2 = xv + out
    m2 = jnp.mean(x2, axis=-1, keepdims=True)
    var2 = jnp.mean((x2 - m2) ** 2, axis=-1, keepdims=True)
    h2 = (x2 - m2) * lax.rsqrt(var2 + 1e-5) * g2_ref[...] + b2_ref[...]
    h2 = jax.nn.gelu(
        jnp.dot(h2, f1w_ref[...], preferred_element_type=jnp.float32)
        + f1b_ref[...]
    )
    h2 = jnp.dot(h2, f2w_ref[...], preferred_element_type=jnp.float32) + f2b_ref[...]
    o_ref[...] = x2 + h2


def _main(xf, agg3, gate_W, gate_b, lin_W3, lin_self_b, self_W, proj_W,
          proj_b, ln1_g, ln1_b, ln2_g, ln2_b, fc1_W, fc1_b, fc2_W, fc2_b,
          bm=2000):
    grid = N // bm

    def full(shape):
        nd = len(shape)
        return pl.BlockSpec(shape, lambda i, _nd=nd: (0,) * _nd)

    return pl.pallas_call(
        _main_body,
        grid=(grid,),
        in_specs=[
            pl.BlockSpec((bm, C), lambda i: (i, 0)),
            pl.BlockSpec((R, bm, C), lambda i: (0, i, 0)),
            full((C, R)),
            full((1, R)),
            full((R, C, C)),
            full((1, C)),
            full((C, C)),
            full((C, C)),
            full((1, C)),
            full((1, C)),
            full((1, C)),
            full((1, C)),
            full((1, C)),
            full((C, HID)),
            full((1, HID)),
            full((HID, C)),
            full((1, C)),
        ],
        out_specs=pl.BlockSpec((bm, C), lambda i: (i, 0)),
        out_shape=jax.ShapeDtypeStruct((N, C), jnp.float32),
    )(xf, agg3, gate_W, gate_b.reshape(1, R), lin_W3,
      lin_self_b.reshape(1, C), self_W, proj_W, proj_b.reshape(1, C),
      ln1_g.reshape(1, C), ln1_b.reshape(1, C), ln2_g.reshape(1, C),
      ln2_b.reshape(1, C), fc1_W, fc1_b.reshape(1, HID), fc2_W,
      fc2_b.reshape(1, C))


def kernel(x, edge_index, edge_type, H, W, ln1_g, ln1_b, gate_W, gate_b,
           lin_W, lin_b, self_W, self_b, proj_W, proj_b, ln2_g, ln2_b,
           fc1_W, fc1_b, fc2_W, fc2_b):
    xf = x.reshape(N, C)
    h = _ln(xf, ln1_g, ln1_b)

    pk = (edge_index[0]
          | (edge_index[1] << 14)
          | (edge_type.astype(jnp.int32) << 28))
    pk_p = jnp.concatenate(
        [pk, jnp.full((E_PAD - E,), jnp.int32(PADV))])

    agg3 = _sc_agg(h, pk_p)

    lin_W3 = lin_W.reshape(R, C, C)
    out = _main(xf, agg3, gate_W, gate_b, lin_W3, lin_b + self_b, self_W,
                proj_W, proj_b, ln1_g, ln1_b, ln2_g, ln2_b, fc1_W, fc1_b,
                fc2_W, fc2_b)
    return out.reshape(1, N, C)


# speedup vs baseline: 8.4525x; 1.0245x over previous
import functools

import jax
import jax.numpy as jnp
from jax import lax
from jax.experimental import pallas as pl
from jax.experimental.pallas import tpu as pltpu
from jax.experimental.pallas import tpu_sc as plsc

N = 10000
C = 128
R = 4
E = 320000
HID = 512
NT = 16
SB = 2048
NSB = 10
EPT = SB * NSB
E_PAD = EPT * NT
DUMP = 240
AGG_ROWS = N + DUMP
RPT = AGG_ROWS // NT
BK = 96
CAP = 2208
DGR = AGG_ROWS // 128
PADV = (3 << 28) | (16383 << 14)


def _ln_body(x_ref, g_ref, b_ref, o_ref):
    xv = x_ref[...]
    m = jnp.mean(xv, axis=-1, keepdims=True)
    var = jnp.mean((xv - m) ** 2, axis=-1, keepdims=True)
    o_ref[...] = (xv - m) * lax.rsqrt(var + 1e-5) * g_ref[...] + b_ref[...]


def _ln(xf, g, b, bm=2000):
    return pl.pallas_call(
        _ln_body,
        grid=(N // bm,),
        in_specs=[
            pl.BlockSpec((bm, C), lambda i: (i, 0)),
            pl.BlockSpec((1, C), lambda i: (0, 0)),
            pl.BlockSpec((1, C), lambda i: (0, 0)),
        ],
        out_specs=pl.BlockSpec((bm, C), lambda i: (i, 0)),
        out_shape=jax.ShapeDtypeStruct((N, C), jnp.float32),
    )(xf, g.reshape(1, C), b.reshape(1, C))


def _pack_body(ei_ref, et_ref, o_ref):
    i = pl.program_id(0)
    src = ei_ref[0]
    dst = ei_ref[1]
    et = et_ref[...]
    pk = src | (dst << 14) | (et << 28)
    rows = ei_ref.shape[1]
    gr = lax.broadcasted_iota(jnp.int32, et.shape, 0) + i * rows
    o_ref[...] = jnp.where(gr < E // 128, pk, PADV)


def _pack(ei2, et2, bm=512):
    grid = (E_PAD // 128) // bm
    return pl.pallas_call(
        _pack_body,
        grid=(grid,),
        in_specs=[
            pl.BlockSpec((2, bm, 128), lambda i: (0, i, 0)),
            pl.BlockSpec((bm, 128), lambda i: (i, 0)),
        ],
        out_specs=pl.BlockSpec((bm, 128), lambda i: (i, 0)),
        out_shape=jax.ShapeDtypeStruct((E_PAD // 128, 128), jnp.int32),
    )(ei2, et2)


def _sc_agg_kernel(h_hbm, pk_hbm, out_hbm,
                   pk_v, csrc_v, clidx_v, lidx_v, gsrc_v, rows_v, ldeg_v,
                   diota_v, gidx_v, dacc_v, agg_sh, deg_sh,
                   gsem0, gsem1, psem):
    c = lax.axis_index("c")
    s = lax.axis_index("s")
    ebase = s * EPT
    lane = lax.iota(jnp.int32, 16)
    dump_v = N + jnp.remainder(s * 16 + lane, DUMP)
    safe_v = s * 16 + lane
    zvec = jnp.zeros((16,), jnp.float32)
    sems = (gsem0, gsem1)

    for v in range(DGR // 16):
        diota_v[0, pl.ds(v * 16, 16)] = v * 16 + lane
    gidx_v[0, pl.ds(0, 16)] = s * (DGR // NT) + jnp.minimum(lane, DGR // NT - 1)

    def issue(b, buf):
        for j in range(BK // 16):
            tv = clidx_v[pl.ds(b * BK + j * 16, 16)]
            sv = csrc_v[pl.ds(b * BK + j * 16, 16)]
            lidx_v[buf, pl.ds(j * 16, 16)] = tv
            gsrc_v[buf, pl.ds(j * 16, 16)] = sv
            cnts, lastm = plsc.scan_count(tv)
            plsc.addupdate_scatter(
                ldeg_v, [tv >> 7, tv & 127], cnts.astype(jnp.float32),
                mask=lastm)
        pltpu.async_copy(
            h_hbm.at[gsrc_v.at[buf]], rows_v.at[buf], sems[buf])

    def wait_gather(buf):
        pltpu.make_async_copy(
            h_hbm.at[gsrc_v.at[buf]], rows_v.at[buf], sems[buf]).wait()

    def scatter(buf):
        pltpu.sync_copy(rows_v.at[buf], agg_sh.at[lidx_v.at[buf]], add=True)

    def fire(nb_):
        @pl.when(nb_ > 0)
        def _():
            issue(0, 0)

            def pair(k, carry2):
                b = 2 * k
                wait_gather(0)
                @pl.when(b + 1 < nb_)
                def _():
                    issue(b + 1, 1)
                scatter(0)
                @pl.when(b + 1 < nb_)
                def _():
                    wait_gather(1)
                    @pl.when(b + 2 < nb_)
                    def __():
                        issue(b + 2, 0)
                    scatter(1)
                return carry2

            lax.fori_loop(0, (nb_ + 1) // 2, pair, 0)

    for p in range(2):
        rbase = (c * 2 + p) * N

        def zero_body(i, carry):
            for j in range(C // 16):
                rows_v[0, i, pl.ds(j * 16, 16)] = zvec
            return carry
        lax.fori_loop(0, BK, zero_body, 0)

        def ldeg_zero(i, carry):
            for j in range(C // 16):
                ldeg_v[i, pl.ds(j * 16, 16)] = zvec
            return carry
        lax.fori_loop(0, DGR, ldeg_zero, 0)

        for j in range(RPT // BK):
            pltpu.sync_copy(rows_v.at[0],
                            agg_sh.at[pl.ds(s * RPT + j * BK, BK)])
        pltpu.sync_copy(
            rows_v.at[0, pl.ds(0, RPT % BK)],
            agg_sh.at[pl.ds(s * RPT + (RPT // BK) * BK, RPT % BK)])
        @pl.when(s < DGR // 8)
        def _():
            pltpu.sync_copy(rows_v.at[0, pl.ds(0, 8)],
                            deg_sh.at[pl.ds(s * 8, 8)])
        plsc.subcore_barrier()

        pltpu.async_copy(pk_hbm.at[pl.ds(ebase, SB)], pk_v.at[0], psem)

        def sb_body(sb, rem):
            buf = sb % 2
            pltpu.make_async_copy(
                pk_hbm.at[pl.ds(ebase, SB)], pk_v.at[buf], psem).wait()

            @pl.when(sb + 1 < NSB)
            def _():
                pltpu.async_copy(
                    pk_hbm.at[pl.ds(ebase + (sb + 1) * SB, SB)],
                    pk_v.at[1 - buf], psem)

            def cvreg(v, cur):
                pk = pk_v[buf, pl.ds(v * 16, 16)]
                sv = pk & 0x3FFF
                dv = (pk >> 14) & 0x3FFF
                ev = pk >> 28
                t = ev * N + dv - rbase
                m = (t >= 0) & (t < N)
                plsc.store_compressed(clidx_v.at[pl.ds(cur, 16)], t, mask=m)
                plsc.store_compressed(csrc_v.at[pl.ds(cur, 16)], sv, mask=m)
                return cur + jnp.sum(m.astype(jnp.int32))

            tot = lax.fori_loop(0, SB // 16, cvreg, rem)
            nb = tot // BK
            fire(nb)
            for j in range(BK // 16):
                t1 = clidx_v[pl.ds(nb * BK + j * 16, 16)]
                t2 = csrc_v[pl.ds(nb * BK + j * 16, 16)]
                clidx_v[pl.ds(j * 16, 16)] = t1
                csrc_v[pl.ds(j * 16, 16)] = t2
            return tot - nb * BK

        rem = lax.fori_loop(0, NSB, sb_body, jnp.int32(0))
        for j in range(BK // 16):
            mpad = (j * 16 + lane) >= rem
            t1 = clidx_v[pl.ds(j * 16, 16)]
            t2 = csrc_v[pl.ds(j * 16, 16)]
            clidx_v[pl.ds(j * 16, 16)] = jnp.where(mpad, dump_v, t1)
            csrc_v[pl.ds(j * 16, 16)] = jnp.where(mpad, safe_v, t2)
        fire(1)
        pltpu.sync_copy(ldeg_v, deg_sh.at[diota_v.at[0]], add=True)
        plsc.subcore_barrier()

        pltpu.async_copy(deg_sh.at[gidx_v.at[0]], dacc_v, psem).wait()

        def norm_chunk(j0, nrows):
            pltpu.sync_copy(agg_sh.at[pl.ds(s * RPT + j0, nrows)],
                            rows_v.at[0, pl.ds(0, nrows)])
            for g in range(nrows // 16):
                w = j0 + g * 16
                dvec = dacc_v[w // 128, pl.ds(w % 128, 16)]
                inv = 1.0 / jnp.maximum(dvec, 1.0)
                for r16 in range(16):
                    sc = jnp.sum(jnp.where(lane == r16, inv, 0.0))
                    row = g * 16 + r16
                    for j in range(C // 16):
                        rows_v[0, row, pl.ds(j * 16, 16)] = (
                            rows_v[0, row, pl.ds(j * 16, 16)] * sc)
            pltpu.sync_copy(
                rows_v.at[0, pl.ds(0, nrows)],
                out_hbm.at[c * 2 + p, pl.ds(s * RPT + j0, nrows)])

        def nbody(jj, carry):
            norm_chunk(jj * BK, BK)
            return carry
        lax.fori_loop(0, RPT // BK, nbody, 0)
        norm_chunk((RPT // BK) * BK, RPT % BK)
        plsc.subcore_barrier()


def _sc_agg(h, pk_p):
    mesh = plsc.VectorSubcoreMesh(core_axis_name="c", subcore_axis_name="s")
    kern = functools.partial(
        pl.kernel,
        mesh=mesh,
        compiler_params=pltpu.CompilerParams(needs_layout_passes=False),
        out_type=jax.ShapeDtypeStruct((R, AGG_ROWS, C), jnp.float32),
        scratch_types=[
            pltpu.VMEM((2, SB), jnp.int32),
            pltpu.VMEM((CAP,), jnp.int32),
            pltpu.VMEM((CAP,), jnp.int32),
            pltpu.VMEM((2, BK), jnp.int32),
            pltpu.VMEM((2, BK), jnp.int32),
            pltpu.VMEM((2, BK, C), jnp.float32),
            pltpu.VMEM((DGR, C), jnp.float32),
            pltpu.VMEM((1, DGR), jnp.int32),
            pltpu.VMEM((1, 16), jnp.int32),
            pltpu.VMEM((16, C), jnp.float32),
            pltpu.VMEM_SHARED((AGG_ROWS, C), jnp.float32),
            pltpu.VMEM_SHARED((DGR, C), jnp.float32),
            pltpu.SemaphoreType.DMA,
            pltpu.SemaphoreType.DMA,
            pltpu.SemaphoreType.DMA,
        ],
    )(_sc_agg_kernel)
    return kern(h, pk_p)


def _main_body(x_ref, agg_ref, gw_ref, gb_ref, lw_ref, lsb_ref, sw_ref,
               pw_ref, pb_ref, g1_ref, b1_ref, g2_ref, b2_ref,
               f1w_ref, f1b_ref, f2w_ref, f2b_ref, o_ref):
    xv = x_ref[...]
    m = jnp.mean(xv, axis=-1, keepdims=True)
    var = jnp.mean((xv - m) ** 2, axis=-1, keepdims=True)
    h = (xv - m) * lax.rsqrt(var + 1e-5) * g1_ref[...] + b1_ref[...]
    gate = jax.nn.sigmoid(
        jnp.dot(h, gw_ref[...], preferred_element_type=jnp.float32)
        + gb_ref[...]
    )
    acc = jnp.dot(h, sw_ref[...], preferred_element_type=jnp.float32)
    for r in range(R):
        ar = agg_ref[r] * gate[:, r:r + 1]
        acc = acc + jnp.dot(ar, lw_ref[r], preferred_element_type=jnp.float32)
    out = jax.nn.gelu(acc + lsb_ref[...])
    out = jnp.dot(out, pw_ref[...], preferred_element_type=jnp.float32) + pb_ref[...]
    x2 = xv + out
    m2 = jnp.mean(x2, axis=-1, keepdims=True)
    var2 = jnp.mean((x2 - m2) ** 2, axis=-1, keepdims=True)
    h2 = (x2 - m2) * lax.rsqrt(var2 + 1e-5) * g2_ref[...] + b2_ref[...]
    h2 = jax.nn.gelu(
        jnp.dot(h2, f1w_ref[...], preferred_element_type=jnp.float32)
        + f1b_ref[...]
    )
    h2 = jnp.dot(h2, f2w_ref[...], preferred_element_type=jnp.float32) + f2b_ref[...]
    o_ref[...] = x2 + h2


def _main(xf, agg3, gate_W, gate_b, lin_W3, lin_self_b, self_W, proj_W,
          proj_b, ln1_g, ln1_b, ln2_g, ln2_b, fc1_W, fc1_b, fc2_W, fc2_b,
          bm=2000):
    grid = N // bm

    def full(shape):
        nd = len(shape)
        return pl.BlockSpec(shape, lambda i, _nd=nd: (0,) * _nd)

    return pl.pallas_call(
        _main_body,
        grid=(grid,),
        in_specs=[
            pl.BlockSpec((bm, C), lambda i: (i, 0)),
            pl.BlockSpec((R, bm, C), lambda i: (0, i, 0)),
            full((C, R)),
            full((1, R)),
            full((R, C, C)),
            full((1, C)),
            full((C, C)),
            full((C, C)),
            full((1, C)),
            full((1, C)),
            full((1, C)),
            full((1, C)),
            full((1, C)),
            full((C, HID)),
            full((1, HID)),
            full((HID, C)),
            full((1, C)),
        ],
        out_specs=pl.BlockSpec((bm, C), lambda i: (i, 0)),
        out_shape=jax.ShapeDtypeStruct((N, C), jnp.float32),
    )(xf, agg3, gate_W, gate_b.reshape(1, R), lin_W3,
      lin_self_b.reshape(1, C), self_W, proj_W, proj_b.reshape(1, C),
      ln1_g.reshape(1, C), ln1_b.reshape(1, C), ln2_g.reshape(1, C),
      ln2_b.reshape(1, C), fc1_W, fc1_b.reshape(1, HID), fc2_W,
      fc2_b.reshape(1, C))


def kernel(x, edge_index, edge_type, H, W, ln1_g, ln1_b, gate_W, gate_b,
           lin_W, lin_b, self_W, self_b, proj_W, proj_b, ln2_g, ln2_b,
           fc1_W, fc1_b, fc2_W, fc2_b):
    xf = x.reshape(N, C)
    h = _ln(xf, ln1_g, ln1_b)

    ei2 = edge_index.reshape(2, E // 128, 128)
    et2 = edge_type.astype(jnp.int32).reshape(E // 128, 128)
    pk_p = _pack(ei2, et2).reshape(E_PAD)

    agg3 = _sc_agg(h, pk_p)

    lin_W3 = lin_W.reshape(R, C, C)
    out = _main(xf, agg3, gate_W, gate_b, lin_W3, lin_b + self_b, self_W,
                proj_W, proj_b, ln1_g, ln1_b, ln2_g, ln2_b, fc1_W, fc1_b,
                fc2_W, fc2_b)
    return out.reshape(1, N, C)
